# Initial kernel scaffold; baseline (speedup 1.0000x reference)
import jax
import jax.numpy as jnp
from jax.experimental import pallas as pl


def kernel(x, edge_index, edge_attr, params):
    raise NotImplementedError("write your pallas kernel here")


# speedup vs baseline: 1.7192x; 1.7192x over previous
import functools

import jax
import jax.numpy as jnp
from jax import lax
from jax.experimental import pallas as pl
from jax.experimental.pallas import tpu as pltpu
from jax.experimental.pallas import tpu_sc as plsc

_N = 50000
_E = 800000
_T = 20
_DIAS = (2, 4, 8)
_C = 16
_TCAT = 66

_NB = 2000
_NBA = 200
_G = _N // _NB
_GA = _N // _NBA

_EPAD = 819200
_EPW = _EPAD // 32
_CH = 128
_NCHUNK = _EPW // _CH
_NTAB = 51200
_ZROWS = 640
_ROWS_OUT = _NTAB // 16

_f32 = jnp.float32


def _relu(v):
    return jnp.maximum(v, 0.0)


def _stats(sums, count):
    s = jnp.sum(sums, axis=(0, 1))
    m = s[0] / count
    v = s[1] / count - m * m
    return jnp.stack([m, lax.rsqrt(v + 1e-5)]).reshape(1, 2)


def _tc_in_body(x_ref, w_ref, b_ref, u_ref, s_ref):
    xb = x_ref[...]
    u = _relu(xb[:, :, None] * w_ref[0][None, None, :]
              + b_ref[0][None, None, :])
    u_ref[...] = u
    s1 = jnp.sum(u, axis=(0, 2))
    s2 = jnp.sum(u * u, axis=(0, 2))
    s_ref[0] = jnp.stack([s1, s2], axis=-1)


def _tc_in(x, w, b):
    return pl.pallas_call(
        _tc_in_body,
        grid=(_G,),
        in_specs=[
            pl.BlockSpec((_NB, _T), lambda i: (i, 0)),
            pl.BlockSpec((1, _C), lambda i: (0, 0)),
            pl.BlockSpec((1, _C), lambda i: (0, 0)),
        ],
        out_specs=[
            pl.BlockSpec((_NB, _T, _C), lambda i: (i, 0, 0)),
            pl.BlockSpec((1, _T, 2), lambda i: (i, 0, 0)),
        ],
        out_shape=[
            jax.ShapeDtypeStruct((_N, _T, _C), _f32),
            jax.ShapeDtypeStruct((_G, _T, 2), _f32),
        ],
    )(x, w, b)


def _tc_aspp_body(u_ref, m_ref, r_ref, lw_ref, lb_ref,
                  w02, w12, bb2, w04, w14, bb4, w08, w18, bb8,
                  ow_ref, ob_ref, h_ref, s_ref):
    u = u_ref[...]
    m = m_ref[0]
    r = r_ref[0]
    hn = ((u - m[None, :, None]) * r[None, :, None]
          * lw_ref[...][:, None, :] + lb_ref[...][:, None, :])
    parts = [hn]
    for d, w0, w1, bb in ((2, w02, w12, bb2), (4, w04, w14, bb4),
                          (8, w08, w18, bb8)):
        tl = _T - d
        a = hn[:, :tl, :].reshape(_NBA * tl, _C)
        c = hn[:, d:, :].reshape(_NBA * tl, _C)
        y = (jnp.dot(a, w0[...], preferred_element_type=_f32, precision=lax.Precision.HIGHEST)
             + jnp.dot(c, w1[...], preferred_element_type=_f32, precision=lax.Precision.HIGHEST)
             + bb[0][None, :])
        parts.append(_relu(y).reshape(_NBA, tl, _C))
    hc = jnp.concatenate(parts, axis=1).reshape(_NBA, _TCAT * _C)
    o = _relu(jnp.dot(hc, ow_ref[...], preferred_element_type=_f32, precision=lax.Precision.HIGHEST)
              + ob_ref[0][None, :])
    h_ref[...] = o
    s_ref[...] = jnp.stack([jnp.sum(o), jnp.sum(o * o)]).reshape(1, 1, 2)


def _tc_aspp(u, mt, rt, lw, lb, tw, ow, ob):
    sm16 = pl.BlockSpec((_C, _C), lambda i: (0, 0))
    b16 = pl.BlockSpec((1, _C), lambda i: (0, 0))
    return pl.pallas_call(
        _tc_aspp_body,
        grid=(_GA,),
        in_specs=[
            pl.BlockSpec((_NBA, _T, _C), lambda i: (i, 0, 0)),
            pl.BlockSpec((1, _T), lambda i: (0, 0)),
            pl.BlockSpec((1, _T), lambda i: (0, 0)),
            pl.BlockSpec((_NBA, _C), lambda i: (i, 0)),
            pl.BlockSpec((_NBA, _C), lambda i: (i, 0)),
            sm16, sm16, b16, sm16, sm16, b16, sm16, sm16, b16,
            pl.BlockSpec((_TCAT * _C, 32), lambda i: (0, 0)),
            pl.BlockSpec((1, 32), lambda i: (0, 0)),
        ],
        out_specs=[
            pl.BlockSpec((_NBA, 32), lambda i: (i, 0)),
            pl.BlockSpec((1, 1, 2), lambda i: (i, 0, 0)),
        ],
        out_shape=[
            jax.ShapeDtypeStruct((_N, 32), _f32),
            jax.ShapeDtypeStruct((_GA, 1, 2), _f32),
        ],
    )(u, mt, rt, lw, lb, *tw, ow, ob)


def _tc_msg_body(split, h_ref, sc_ref, lw_ref, lb_ref, wm_ref, ws_ref, *outs):
    m = sc_ref[0, 0]
    r = sc_ref[0, 1]
    h = (h_ref[...] - m) * r * lw_ref[...] + lb_ref[...]
    z = jnp.dot(h, wm_ref[...], preferred_element_type=_f32, precision=lax.Precision.HIGHEST)
    s = jnp.dot(h, ws_ref[...], preferred_element_type=_f32, precision=lax.Precision.HIGHEST)
    if split:
        outs[0][...] = z[:, :32]
        outs[1][...] = z[:, 32:]
        outs[2][...] = s
    else:
        outs[0][...] = z
        outs[1][...] = s


def _tc_msg(hpre, scal, lw, lb, wm, ws, cin, cout):
    split = cout == 64
    zw = 32 if split else cout
    zspecs = [pl.BlockSpec((_NB, zw), lambda i: (i, 0))] * (2 if split else 1)
    zshapes = [jax.ShapeDtypeStruct((_N, zw), _f32)] * (2 if split else 1)
    return pl.pallas_call(
        functools.partial(_tc_msg_body, split),
        grid=(_G,),
        in_specs=[
            pl.BlockSpec((_NB, cin), lambda i: (i, 0)),
            pl.BlockSpec((1, 2), lambda i: (0, 0)),
            pl.BlockSpec((_NB, cin), lambda i: (i, 0)),
            pl.BlockSpec((_NB, cin), lambda i: (i, 0)),
            pl.BlockSpec((cin, cout), lambda i: (0, 0)),
            pl.BlockSpec((cin, cout), lambda i: (0, 0)),
        ],
        out_specs=zspecs + [pl.BlockSpec((_NB, cout), lambda i: (i, 0))],
        out_shape=zshapes + [jax.ShapeDtypeStruct((_N, cout), _f32)],
    )(hpre, scal, lw, lb, wm, ws)


def _tc_comb_body(split, s_ref, *refs):
    if split:
        agg_a, agg_b, a2_ref, we_ref, b_ref, h_ref, sums_ref = refs
        agg = jnp.concatenate([agg_a[0] + agg_a[1], agg_b[0] + agg_b[1]],
                              axis=1)
    else:
        agg_a, a2_ref, we_ref, b_ref, h_ref, sums_ref = refs
        agg = agg_a[0] + agg_a[1]
    a = a2_ref[0] + a2_ref[1]
    ea = jnp.dot(a, we_ref[...], preferred_element_type=_f32, precision=lax.Precision.HIGHEST)
    h = _relu(s_ref[...] + agg + ea + b_ref[0][None, :])
    h_ref[...] = h
    sums_ref[...] = jnp.stack([jnp.sum(h), jnp.sum(h * h)]).reshape(1, 1, 2)


def _tc_comb(s, aggs, a2, wep, b, cout):
    split = cout == 64
    zw = 32 if split else cout
    aspec = [pl.BlockSpec((2, _NB, zw), lambda i: (0, i, 0))] * len(aggs)
    return pl.pallas_call(
        functools.partial(_tc_comb_body, split),
        grid=(_G,),
        in_specs=[pl.BlockSpec((_NB, cout), lambda i: (i, 0))] + aspec + [
            pl.BlockSpec((2, _NB, 8), lambda i: (0, i, 0)),
            pl.BlockSpec((8, cout), lambda i: (0, 0)),
            pl.BlockSpec((1, cout), lambda i: (0, 0)),
        ],
        out_specs=[
            pl.BlockSpec((_NB, cout), lambda i: (i, 0)),
            pl.BlockSpec((1, 1, 2), lambda i: (i, 0, 0)),
        ],
        out_shape=[
            jax.ShapeDtypeStruct((_N, cout), _f32),
            jax.ShapeDtypeStruct((_G, 1, 2), _f32),
        ],
    )(s, *aggs, a2, wep, b)


def _tc_out1_body(h_ref, sc_ref, lw_ref, lb_ref, w_ref, b_ref, o_ref, s_ref):
    m = sc_ref[0, 0]
    r = sc_ref[0, 1]
    h = (h_ref[...] - m) * r * lw_ref[...] + lb_ref[...]
    o = jnp.dot(h, w_ref[...], preferred_element_type=_f32, precision=lax.Precision.HIGHEST) + b_ref[0][None, :]
    o_ref[...] = o
    s_ref[...] = jnp.stack([jnp.sum(o), jnp.sum(o * o)]).reshape(1, 1, 2)


def _tc_out1(hpre, scal, lw, lb, w1, b1):
    return pl.pallas_call(
        _tc_out1_body,
        grid=(_G,),
        in_specs=[
            pl.BlockSpec((_NB, _C), lambda i: (i, 0)),
            pl.BlockSpec((1, 2), lambda i: (0, 0)),
            pl.BlockSpec((_NB, _C), lambda i: (i, 0)),
            pl.BlockSpec((_NB, _C), lambda i: (i, 0)),
            pl.BlockSpec((_C, _C), lambda i: (0, 0)),
            pl.BlockSpec((1, _C), lambda i: (0, 0)),
        ],
        out_specs=[
            pl.BlockSpec((_NB, _C), lambda i: (i, 0)),
            pl.BlockSpec((1, 1, 2), lambda i: (i, 0, 0)),
        ],
        out_shape=[
            jax.ShapeDtypeStruct((_N, _C), _f32),
            jax.ShapeDtypeStruct((_G, 1, 2), _f32),
        ],
    )(hpre, scal, lw, lb, w1, b1)


def _tc_out2_body(h_ref, sc_ref, lw_ref, lb_ref, w2_ref, b2_ref,
                  wf_ref, bf_ref, o_ref):
    m = sc_ref[0, 0]
    r = sc_ref[0, 1]
    h = (h_ref[...] - m) * r * lw_ref[...] + lb_ref[...]
    h = jnp.dot(h, w2_ref[...], preferred_element_type=_f32, precision=lax.Precision.HIGHEST) + b2_ref[0][None, :]
    o_ref[...] = (jnp.dot(h, wf_ref[...], preferred_element_type=_f32, precision=lax.Precision.HIGHEST)
                  + bf_ref[0][None, :])


def _tc_out2(o1, scal, lw, lb, w2, b2, wf, bf):
    return pl.pallas_call(
        _tc_out2_body,
        grid=(_G,),
        in_specs=[
            pl.BlockSpec((_NB, _C), lambda i: (i, 0)),
            pl.BlockSpec((1, 2), lambda i: (0, 0)),
            pl.BlockSpec((_NB, _C), lambda i: (i, 0)),
            pl.BlockSpec((_NB, _C), lambda i: (i, 0)),
            pl.BlockSpec((_C, _C), lambda i: (0, 0)),
            pl.BlockSpec((1, _C), lambda i: (0, 0)),
            pl.BlockSpec((_C, 8), lambda i: (0, 0)),
            pl.BlockSpec((1, 8), lambda i: (0, 0)),
        ],
        out_specs=pl.BlockSpec((_NB, 8), lambda i: (i, 0)),
        out_shape=jax.ShapeDtypeStruct((_N, 8), _f32),
    )(o1, scal, lw, lb, w2, b2, wf, bf)


def _make_sc_scatter(n_tables, d):
    mesh = plsc.VectorSubcoreMesh(core_axis_name="c", subcore_axis_name="s")
    out_type = [jax.ShapeDtypeStruct((2, _NTAB, d), _f32)
                for _ in range(n_tables)]
    scratch = [
        pltpu.VMEM((_CH,), jnp.int32),
        pltpu.VMEM((_CH,), jnp.int32),
        pltpu.VMEM((_CH, d), _f32),
        pltpu.VMEM((_ZROWS, d), _f32),
        pltpu.VMEM_SHARED((_NTAB, d), _f32),
        pltpu.SemaphoreType.DMA,
    ]

    @functools.partial(pl.kernel, out_type=out_type, mesh=mesh,
                       scratch_types=scratch,
                       compiler_params=pltpu.CompilerParams(
                           use_tc_tiling_on_sc=False))
    def sc_kernel(*refs):
        tables = refs[:n_tables]
        src_hbm, dst_hbm, z_hbm = refs[n_tables:n_tables + 3]
        outs = refs[n_tables + 3:2 * n_tables + 3]
        src_v, dst_v, rows_v, zero_v, tab_sh, sem = refs[2 * n_tables + 3:]
        cid = lax.axis_index("c")
        sid = lax.axis_index("s")
        wid = cid * 16 + sid
        ebase = wid * _EPW
        pltpu.sync_copy(z_hbm, zero_v)

        for p in range(n_tables):
            def zbody(i, _, sid=sid):
                off = pl.multiple_of(sid * 3200 + i * _ZROWS, 8)
                pltpu.sync_copy(zero_v, tab_sh.at[pl.ds(off, _ZROWS)])
                return 0
            lax.fori_loop(0, 3200 // _ZROWS, zbody, 0)
            plsc.subcore_barrier()

            def cbody(k, _, tab=tables[p]):
                base = pl.multiple_of(ebase + k * _CH, 8)
                pltpu.sync_copy(src_hbm.at[pl.ds(base, _CH)], src_v)
                pltpu.sync_copy(dst_hbm.at[pl.ds(base, _CH)], dst_v)
                pltpu.async_copy(tab.at[src_v], rows_v, sem).wait()
                pltpu.sync_copy(rows_v, tab_sh.at[dst_v], add=True)
                return 0
            lax.fori_loop(0, _NCHUNK, cbody, 0)
            plsc.subcore_barrier()

            off = pl.multiple_of(sid * _ROWS_OUT, 8)
            pltpu.sync_copy(tab_sh.at[pl.ds(off, _ROWS_OUT)],
                            outs[p].at[cid, pl.ds(off, _ROWS_OUT)])
            if p < n_tables - 1:
                plsc.subcore_barrier()

    return sc_kernel


_sc_cache = {}


def _sc_scatter(n_tables, d, *args):
    key = (n_tables, d)
    if key not in _sc_cache:
        _sc_cache[key] = _make_sc_scatter(n_tables, d)
    return _sc_cache[key](*args)


def kernel(x, edge_index, edge_attr, params):
    p = params

    inw = p['in_W'].reshape(1, _C)
    inb = p['in_b'].reshape(1, _C)
    tw = []
    for d in _DIAS:
        w = p['t%d_W' % d]
        tw += [w[:, :, 0].T, w[:, :, 1].T, p['t%d_b' % d].reshape(1, _C)]
    ow = p['out_W'].transpose(2, 1, 0).reshape(_TCAT * _C, 32)
    ob = p['out_b'].reshape(1, 32)

    npad = _EPAD - _E
    padi = jnp.arange(npad, dtype=jnp.int32)
    src_p = jnp.concatenate([edge_index[0], padi % jnp.int32(_N)])
    dst_p = jnp.concatenate([edge_index[1], jnp.int32(_N) + (padi % 1024)])
    iota_e = jnp.arange(_EPAD, dtype=jnp.int32)
    attr8 = jnp.zeros((_EPAD, 8), _f32).at[:_E, :4].set(edge_attr)
    z32 = jnp.zeros((_ZROWS, 32), _f32)
    z16 = jnp.zeros((_ZROWS, 16), _f32)
    z8 = jnp.zeros((_ZROWS, 8), _f32)

    (a2,) = _sc_scatter(1, 8, attr8, iota_e, dst_p, z8)

    u, sums_t = _tc_in(x, inw, inb)
    st = jnp.sum(sums_t, axis=0)
    cnt = jnp.float32(_N * _C)
    mt = st[:, 0] / cnt
    vt = st[:, 1] / cnt - mt * mt
    h1pre, s1 = _tc_aspp(u, mt.reshape(1, _T),
                         lax.rsqrt(vt + 1e-5).reshape(1, _T),
                         p['ln_in_w'], p['ln_in_b'], tw, ow, ob)

    scal = _stats(s1, _N * 32.0)
    z0a, z0b, s0 = _tc_msg(h1pre, scal, p['ln_out_w'], p['ln_out_b'],
                           p['s0_Wmsg'], p['s0_Wself'], 32, 64)
    agg0a, agg0b = _sc_scatter(2, 32, z0a, z0b, src_p, dst_p, z32)
    wep0 = jnp.zeros((8, 64), _f32).at[:4].set(p['s0_We'])
    h2pre, s2 = _tc_comb(s0, [agg0a, agg0b], a2, wep0,
                         p['s0_b'].reshape(1, 64), 64)

    scal = _stats(s2, _N * 64.0)
    z1, s1s = _tc_msg(h2pre, scal, p['n0_w'], p['n0_b'],
                      p['s1_Wmsg'], p['s1_Wself'], 64, 32)
    (agg1,) = _sc_scatter(1, 32, z1, src_p, dst_p, z32)
    wep1 = jnp.zeros((8, 32), _f32).at[:4].set(p['s1_We'])
    h3pre, s3 = _tc_comb(s1s, [agg1], a2, wep1, p['s1_b'].reshape(1, 32), 32)

    scal = _stats(s3, _N * 32.0)
    z2, s2s = _tc_msg(h3pre, scal, p['n1_w'], p['n1_b'],
                      p['s2_Wmsg'], p['s2_Wself'], 32, 16)
    (agg2,) = _sc_scatter(1, 16, z2, src_p, dst_p, z16)
    wep2 = jnp.zeros((8, 16), _f32).at[:4].set(p['s2_We'])
    h4pre, s4 = _tc_comb(s2s, [agg2], a2, wep2, p['s2_b'].reshape(1, 16), 16)

    scal = _stats(s4, _N * 16.0)
    o1pre, s5 = _tc_out1(h4pre, scal, p['n2_w'], p['n2_b'],
                         p['o_W1'], p['o_b1'].reshape(1, _C))
    scal = _stats(s5, _N * 16.0)
    wf = jnp.zeros((_C, 8), _f32).at[:, :1].set(p['o_Wfc'])
    bf = jnp.zeros((1, 8), _f32).at[0, 0].set(p['o_bfc'][0])
    o = _tc_out2(o1pre, scal, p['o_lnw'], p['o_lnb'],
                 p['o_W2'], p['o_b2'].reshape(1, _C), wf, bf)
    return o[:, :1].reshape(1, _N, 1)


# speedup vs baseline: 1.8339x; 1.0667x over previous
import functools

import jax
import jax.numpy as jnp
from jax import lax
from jax.experimental import pallas as pl
from jax.experimental.pallas import tpu as pltpu
from jax.experimental.pallas import tpu_sc as plsc

_N = 50000
_E = 800000
_T = 20
_DIAS = (2, 4, 8)
_C = 16
_TCAT = 66

_NB = 2000
_NBA = 200
_G = _N // _NB
_GA = _N // _NBA

_EPAD = 819200
_EPW = _EPAD // 32
_CH = 128
_NCHUNK = _EPW // _CH
_NTAB = 51200
_ZROWS = 128
_ROWS_OUT = _NTAB // 16

_f32 = jnp.float32


def _relu(v):
    return jnp.maximum(v, 0.0)


def _stats(sums, count):
    s = jnp.sum(sums, axis=(0, 1))
    m = s[0] / count
    v = s[1] / count - m * m
    return jnp.stack([m, lax.rsqrt(v + 1e-5)]).reshape(1, 2)


def _tc_in_body(x_ref, w_ref, b_ref, u_ref, s_ref):
    xb = x_ref[...]
    u = _relu(xb[:, :, None] * w_ref[0][None, None, :]
              + b_ref[0][None, None, :])
    u_ref[...] = u
    s1 = jnp.sum(u, axis=(0, 2))
    s2 = jnp.sum(u * u, axis=(0, 2))
    s_ref[0] = jnp.stack([s1, s2], axis=-1)


def _tc_in(x, w, b):
    return pl.pallas_call(
        _tc_in_body,
        grid=(_G,),
        in_specs=[
            pl.BlockSpec((_NB, _T), lambda i: (i, 0)),
            pl.BlockSpec((1, _C), lambda i: (0, 0)),
            pl.BlockSpec((1, _C), lambda i: (0, 0)),
        ],
        out_specs=[
            pl.BlockSpec((_NB, _T, _C), lambda i: (i, 0, 0)),
            pl.BlockSpec((1, _T, 2), lambda i: (i, 0, 0)),
        ],
        out_shape=[
            jax.ShapeDtypeStruct((_N, _T, _C), _f32),
            jax.ShapeDtypeStruct((_G, _T, 2), _f32),
        ],
    )(x, w, b)


def _tc_aspp_body(u_ref, m_ref, r_ref, lw_ref, lb_ref,
                  w02, w12, bb2, w04, w14, bb4, w08, w18, bb8,
                  ow_ref, ob_ref, h_ref, s_ref):
    u = u_ref[...]
    m = m_ref[0]
    r = r_ref[0]
    hn = ((u - m[None, :, None]) * r[None, :, None]
          * lw_ref[...][:, None, :] + lb_ref[...][:, None, :])
    parts = [hn]
    for d, w0, w1, bb in ((2, w02, w12, bb2), (4, w04, w14, bb4),
                          (8, w08, w18, bb8)):
        tl = _T - d
        a = hn[:, :tl, :].reshape(_NBA * tl, _C)
        c = hn[:, d:, :].reshape(_NBA * tl, _C)
        y = (jnp.dot(a, w0[...], preferred_element_type=_f32, precision=lax.Precision.HIGHEST)
             + jnp.dot(c, w1[...], preferred_element_type=_f32, precision=lax.Precision.HIGHEST)
             + bb[0][None, :])
        parts.append(_relu(y).reshape(_NBA, tl, _C))
    hc = jnp.concatenate(parts, axis=1).reshape(_NBA, _TCAT * _C)
    o = _relu(jnp.dot(hc, ow_ref[...], preferred_element_type=_f32, precision=lax.Precision.HIGHEST)
              + ob_ref[0][None, :])
    h_ref[...] = o
    s_ref[...] = jnp.stack([jnp.sum(o), jnp.sum(o * o)]).reshape(1, 1, 2)


def _tc_aspp(u, mt, rt, lw, lb, tw, ow, ob):
    sm16 = pl.BlockSpec((_C, _C), lambda i: (0, 0))
    b16 = pl.BlockSpec((1, _C), lambda i: (0, 0))
    return pl.pallas_call(
        _tc_aspp_body,
        grid=(_GA,),
        in_specs=[
            pl.BlockSpec((_NBA, _T, _C), lambda i: (i, 0, 0)),
            pl.BlockSpec((1, _T), lambda i: (0, 0)),
            pl.BlockSpec((1, _T), lambda i: (0, 0)),
            pl.BlockSpec((_NBA, _C), lambda i: (i, 0)),
            pl.BlockSpec((_NBA, _C), lambda i: (i, 0)),
            sm16, sm16, b16, sm16, sm16, b16, sm16, sm16, b16,
            pl.BlockSpec((_TCAT * _C, 32), lambda i: (0, 0)),
            pl.BlockSpec((1, 32), lambda i: (0, 0)),
        ],
        out_specs=[
            pl.BlockSpec((_NBA, 32), lambda i: (i, 0)),
            pl.BlockSpec((1, 1, 2), lambda i: (i, 0, 0)),
        ],
        out_shape=[
            jax.ShapeDtypeStruct((_N, 32), _f32),
            jax.ShapeDtypeStruct((_GA, 1, 2), _f32),
        ],
    )(u, mt, rt, lw, lb, *tw, ow, ob)


def _tc_msg_body(split, h_ref, sc_ref, lw_ref, lb_ref, wm_ref, ws_ref, *outs):
    m = sc_ref[0, 0]
    r = sc_ref[0, 1]
    h = (h_ref[...] - m) * r * lw_ref[...] + lb_ref[...]
    z = jnp.dot(h, wm_ref[...], preferred_element_type=_f32, precision=lax.Precision.HIGHEST)
    s = jnp.dot(h, ws_ref[...], preferred_element_type=_f32, precision=lax.Precision.HIGHEST)
    if split:
        outs[0][...] = z[:, :32]
        outs[1][...] = z[:, 32:]
        outs[2][...] = s
    else:
        outs[0][...] = z
        outs[1][...] = s


def _tc_msg(hpre, scal, lw, lb, wm, ws, cin, cout):
    split = cout == 64
    zw = 32 if split else cout
    zspecs = [pl.BlockSpec((_NB, zw), lambda i: (i, 0))] * (2 if split else 1)
    zshapes = [jax.ShapeDtypeStruct((_N, zw), _f32)] * (2 if split else 1)
    return pl.pallas_call(
        functools.partial(_tc_msg_body, split),
        grid=(_G,),
        in_specs=[
            pl.BlockSpec((_NB, cin), lambda i: (i, 0)),
            pl.BlockSpec((1, 2), lambda i: (0, 0)),
            pl.BlockSpec((_NB, cin), lambda i: (i, 0)),
            pl.BlockSpec((_NB, cin), lambda i: (i, 0)),
            pl.BlockSpec((cin, cout), lambda i: (0, 0)),
            pl.BlockSpec((cin, cout), lambda i: (0, 0)),
        ],
        out_specs=zspecs + [pl.BlockSpec((_NB, cout), lambda i: (i, 0))],
        out_shape=zshapes + [jax.ShapeDtypeStruct((_N, cout), _f32)],
    )(hpre, scal, lw, lb, wm, ws)


def _tc_comb_body(split, s_ref, *refs):
    if split:
        agg_a, agg_b, a2_ref, we_ref, b_ref, h_ref, sums_ref = refs
        agg = jnp.concatenate([agg_a[0] + agg_a[1], agg_b[0] + agg_b[1]],
                              axis=1)
    else:
        agg_a, a2_ref, we_ref, b_ref, h_ref, sums_ref = refs
        agg = agg_a[0] + agg_a[1]
    a = a2_ref[0] + a2_ref[1]
    ea = jnp.dot(a, we_ref[...], preferred_element_type=_f32, precision=lax.Precision.HIGHEST)
    h = _relu(s_ref[...] + agg + ea + b_ref[0][None, :])
    h_ref[...] = h
    sums_ref[...] = jnp.stack([jnp.sum(h), jnp.sum(h * h)]).reshape(1, 1, 2)


def _tc_comb(s, aggs, a2, wep, b, cout):
    split = cout == 64
    zw = 32 if split else cout
    aspec = [pl.BlockSpec((2, _NB, zw), lambda i: (0, i, 0))] * len(aggs)
    return pl.pallas_call(
        functools.partial(_tc_comb_body, split),
        grid=(_G,),
        in_specs=[pl.BlockSpec((_NB, cout), lambda i: (i, 0))] + aspec + [
            pl.BlockSpec((2, _NB, 8), lambda i: (0, i, 0)),
            pl.BlockSpec((8, cout), lambda i: (0, 0)),
            pl.BlockSpec((1, cout), lambda i: (0, 0)),
        ],
        out_specs=[
            pl.BlockSpec((_NB, cout), lambda i: (i, 0)),
            pl.BlockSpec((1, 1, 2), lambda i: (i, 0, 0)),
        ],
        out_shape=[
            jax.ShapeDtypeStruct((_N, cout), _f32),
            jax.ShapeDtypeStruct((_G, 1, 2), _f32),
        ],
    )(s, *aggs, a2, wep, b)


def _tc_out1_body(h_ref, sc_ref, lw_ref, lb_ref, w_ref, b_ref, o_ref, s_ref):
    m = sc_ref[0, 0]
    r = sc_ref[0, 1]
    h = (h_ref[...] - m) * r * lw_ref[...] + lb_ref[...]
    o = jnp.dot(h, w_ref[...], preferred_element_type=_f32, precision=lax.Precision.HIGHEST) + b_ref[0][None, :]
    o_ref[...] = o
    s_ref[...] = jnp.stack([jnp.sum(o), jnp.sum(o * o)]).reshape(1, 1, 2)


def _tc_out1(hpre, scal, lw, lb, w1, b1):
    return pl.pallas_call(
        _tc_out1_body,
        grid=(_G,),
        in_specs=[
            pl.BlockSpec((_NB, _C), lambda i: (i, 0)),
            pl.BlockSpec((1, 2), lambda i: (0, 0)),
            pl.BlockSpec((_NB, _C), lambda i: (i, 0)),
            pl.BlockSpec((_NB, _C), lambda i: (i, 0)),
            pl.BlockSpec((_C, _C), lambda i: (0, 0)),
            pl.BlockSpec((1, _C), lambda i: (0, 0)),
        ],
        out_specs=[
            pl.BlockSpec((_NB, _C), lambda i: (i, 0)),
            pl.BlockSpec((1, 1, 2), lambda i: (i, 0, 0)),
        ],
        out_shape=[
            jax.ShapeDtypeStruct((_N, _C), _f32),
            jax.ShapeDtypeStruct((_G, 1, 2), _f32),
        ],
    )(hpre, scal, lw, lb, w1, b1)


def _tc_out2_body(h_ref, sc_ref, lw_ref, lb_ref, w2_ref, b2_ref,
                  wf_ref, bf_ref, o_ref):
    m = sc_ref[0, 0]
    r = sc_ref[0, 1]
    h = (h_ref[...] - m) * r * lw_ref[...] + lb_ref[...]
    h = jnp.dot(h, w2_ref[...], preferred_element_type=_f32, precision=lax.Precision.HIGHEST) + b2_ref[0][None, :]
    o_ref[...] = (jnp.dot(h, wf_ref[...], preferred_element_type=_f32, precision=lax.Precision.HIGHEST)
                  + bf_ref[0][None, :])


def _tc_out2(o1, scal, lw, lb, w2, b2, wf, bf):
    return pl.pallas_call(
        _tc_out2_body,
        grid=(_G,),
        in_specs=[
            pl.BlockSpec((_NB, _C), lambda i: (i, 0)),
            pl.BlockSpec((1, 2), lambda i: (0, 0)),
            pl.BlockSpec((_NB, _C), lambda i: (i, 0)),
            pl.BlockSpec((_NB, _C), lambda i: (i, 0)),
            pl.BlockSpec((_C, _C), lambda i: (0, 0)),
            pl.BlockSpec((1, _C), lambda i: (0, 0)),
            pl.BlockSpec((_C, 8), lambda i: (0, 0)),
            pl.BlockSpec((1, 8), lambda i: (0, 0)),
        ],
        out_specs=pl.BlockSpec((_NB, 8), lambda i: (i, 0)),
        out_shape=jax.ShapeDtypeStruct((_N, 8), _f32),
    )(o1, scal, lw, lb, w2, b2, wf, bf)


def _tc_pad8_body(a_ref, o_ref):
    o_ref[...] = jnp.concatenate(
        [a_ref[...], jnp.zeros((_E // 100, 4), _f32)], axis=1)


def _tc_pad8(edge_attr):
    return pl.pallas_call(
        _tc_pad8_body,
        grid=(100,),
        in_specs=[pl.BlockSpec((_E // 100, 4), lambda i: (i, 0))],
        out_specs=pl.BlockSpec((_E // 100, 8), lambda i: (i, 0)),
        out_shape=jax.ShapeDtypeStruct((_E, 8), _f32),
    )(edge_attr)


_MAC = 2
_MB = _MAC * _CH
_NMAC = _EPW // _MB


def _make_sc_scatter(n_tables, d):
    mesh = plsc.VectorSubcoreMesh(core_axis_name="c", subcore_axis_name="s")
    out_type = [jax.ShapeDtypeStruct((2, _NTAB, d), _f32)
                for _ in range(n_tables)]
    scratch = [
        pltpu.VMEM((_MAC, _CH), jnp.int32),
        pltpu.VMEM((_MAC, _CH), jnp.int32),
        pltpu.VMEM((_MAC, _CH), jnp.int32),
        pltpu.VMEM((_MAC, _CH), jnp.int32),
        pltpu.VMEM((_MB, d), _f32),
        pltpu.VMEM((_MB, d), _f32),
        pltpu.VMEM((_ZROWS, d), _f32),
        pltpu.VMEM_SHARED((_NTAB, d), _f32),
        pltpu.SemaphoreType.DMA,
        pltpu.SemaphoreType.DMA,
    ]

    @functools.partial(pl.kernel, out_type=out_type, mesh=mesh,
                       scratch_types=scratch,
                       compiler_params=pltpu.CompilerParams(
                           use_tc_tiling_on_sc=False))
    def sc_kernel(*refs):
        tables = refs[:n_tables]
        src_hbm, dst_hbm, z_hbm = refs[n_tables:n_tables + 3]
        outs = refs[n_tables + 3:2 * n_tables + 3]
        (sidx0, sidx1, didx0, didx1, rows0, rows1, zero_v, tab_sh,
         semg0, semg1) = refs[2 * n_tables + 3:]
        cid = lax.axis_index("c")
        sid = lax.axis_index("s")
        wid = cid * 16 + sid
        crow = wid * _NCHUNK
        pltpu.sync_copy(z_hbm, zero_v)

        maxm = _NMAC - 1

        def load_idx(m, sbuf, dbuf):
            r = pl.multiple_of(
                jnp.minimum(crow + m * _MAC, crow + maxm * _MAC), 2)
            pltpu.sync_copy(src_hbm.at[pl.ds(r, _MAC)], sbuf)
            pltpu.sync_copy(dst_hbm.at[pl.ds(r, _MAC)], dbuf)

        def fire(tab, sbuf, rows, sem):
            for j in range(_MAC):
                pltpu.async_copy(tab.at[sbuf.at[j]],
                                 rows.at[pl.ds(j * _CH, _CH)], sem).wait()

        def scatter(dbuf, rows):
            for j in range(_MAC):
                pltpu.sync_copy(rows.at[pl.ds(j * _CH, _CH)],
                                tab_sh.at[dbuf.at[j]], add=True)

        for p in range(n_tables):
            tab = tables[p]

            def zbody(i, _, sid=sid):
                off = pl.multiple_of(sid * 3200 + i * _ZROWS, 8)
                pltpu.sync_copy(zero_v, tab_sh.at[pl.ds(off, _ZROWS)])
                return 0
            lax.fori_loop(0, 3200 // _ZROWS, zbody, 0)
            plsc.subcore_barrier()

            def qbody(m, _, tab=tab):
                load_idx(m, sidx0, didx0)
                fire(tab, sidx0, rows0, semg0)
                scatter(didx0, rows0)
                return 0
            lax.fori_loop(0, _NMAC, qbody, 0)
            plsc.subcore_barrier()

            off = pl.multiple_of(sid * _ROWS_OUT, 8)
            pltpu.sync_copy(tab_sh.at[pl.ds(off, _ROWS_OUT)],
                            outs[p].at[cid, pl.ds(off, _ROWS_OUT)])
            if p < n_tables - 1:
                plsc.subcore_barrier()

    return sc_kernel


_sc_cache = {}


def _sc_scatter(n_tables, d, *args):
    key = (n_tables, d)
    if key not in _sc_cache:
        _sc_cache[key] = _make_sc_scatter(n_tables, d)
    return _sc_cache[key](*args)


def kernel(x, edge_index, edge_attr, params):
    p = params

    inw = p['in_W'].reshape(1, _C)
    inb = p['in_b'].reshape(1, _C)
    tw = []
    for d in _DIAS:
        w = p['t%d_W' % d]
        tw += [w[:, :, 0].T, w[:, :, 1].T, p['t%d_b' % d].reshape(1, _C)]
    ow = p['out_W'].transpose(2, 1, 0).reshape(_TCAT * _C, 32)
    ob = p['out_b'].reshape(1, 32)

    npad = _EPAD - _E
    padi = jnp.arange(npad, dtype=jnp.int32)
    src_p = jnp.concatenate([edge_index[0], padi % jnp.int32(_N)])
    dst_p = jnp.concatenate([edge_index[1], jnp.int32(_N) + (padi % 1024)])
    iota_e = jnp.arange(_EPAD, dtype=jnp.int32) % jnp.int32(_E)
    dst2 = dst_p.reshape(-1, _CH)
    src2 = src_p.reshape(-1, _CH)
    iot2 = iota_e.reshape(-1, _CH)
    attr8 = _tc_pad8(edge_attr)
    z32 = jnp.zeros((_ZROWS, 32), _f32)
    z16 = jnp.zeros((_ZROWS, 16), _f32)
    z8 = jnp.zeros((_ZROWS, 8), _f32)

    (a2,) = _sc_scatter(1, 8, attr8, iot2, dst2, z8)

    u, sums_t = _tc_in(x, inw, inb)
    st = jnp.sum(sums_t, axis=0)
    cnt = jnp.float32(_N * _C)
    mt = st[:, 0] / cnt
    vt = st[:, 1] / cnt - mt * mt
    h1pre, s1 = _tc_aspp(u, mt.reshape(1, _T),
                         lax.rsqrt(vt + 1e-5).reshape(1, _T),
                         p['ln_in_w'], p['ln_in_b'], tw, ow, ob)

    scal = _stats(s1, _N * 32.0)
    z0a, z0b, s0 = _tc_msg(h1pre, scal, p['ln_out_w'], p['ln_out_b'],
                           p['s0_Wmsg'], p['s0_Wself'], 32, 64)
    agg0a, agg0b = _sc_scatter(2, 32, z0a, z0b, src2, dst2, z32)
    wep0 = jnp.zeros((8, 64), _f32).at[:4].set(p['s0_We'])
    h2pre, s2 = _tc_comb(s0, [agg0a, agg0b], a2, wep0,
                         p['s0_b'].reshape(1, 64), 64)

    scal = _stats(s2, _N * 64.0)
    z1, s1s = _tc_msg(h2pre, scal, p['n0_w'], p['n0_b'],
                      p['s1_Wmsg'], p['s1_Wself'], 64, 32)
    (agg1,) = _sc_scatter(1, 32, z1, src2, dst2, z32)
    wep1 = jnp.zeros((8, 32), _f32).at[:4].set(p['s1_We'])
    h3pre, s3 = _tc_comb(s1s, [agg1], a2, wep1, p['s1_b'].reshape(1, 32), 32)

    scal = _stats(s3, _N * 32.0)
    z2, s2s = _tc_msg(h3pre, scal, p['n1_w'], p['n1_b'],
                      p['s2_Wmsg'], p['s2_Wself'], 32, 16)
    (agg2,) = _sc_scatter(1, 16, z2, src2, dst2, z16)
    wep2 = jnp.zeros((8, 16), _f32).at[:4].set(p['s2_We'])
    h4pre, s4 = _tc_comb(s2s, [agg2], a2, wep2, p['s2_b'].reshape(1, 16), 16)

    scal = _stats(s4, _N * 16.0)
    o1pre, s5 = _tc_out1(h4pre, scal, p['n2_w'], p['n2_b'],
                         p['o_W1'], p['o_b1'].reshape(1, _C))
    scal = _stats(s5, _N * 16.0)
    wf = jnp.zeros((_C, 8), _f32).at[:, :1].set(p['o_Wfc'])
    bf = jnp.zeros((1, 8), _f32).at[0, 0].set(p['o_bfc'][0])
    o = _tc_out2(o1pre, scal, p['o_lnw'], p['o_lnb'],
                 p['o_W2'], p['o_b2'].reshape(1, _C), wf, bf)
    return o[:, :1].reshape(1, _N, 1)


# speedup vs baseline: 1.9284x; 1.0515x over previous
import functools

import jax
import jax.numpy as jnp
from jax import lax
from jax.experimental import pallas as pl
from jax.experimental.pallas import tpu as pltpu
from jax.experimental.pallas import tpu_sc as plsc

_N = 50000
_E = 800000
_T = 20
_DIAS = (2, 4, 8)
_C = 16
_TCAT = 66

_NB = 2000
_NBA = 200
_G = _N // _NB
_GA = _N // _NBA

_EPAD = 819200
_EPW = _EPAD // 32
_CH = 128
_NCHUNK = _EPW // _CH
_NTAB = 51200
_ZROWS = 128
_ROWS_OUT = _NTAB // 16

_f32 = jnp.float32


def _relu(v):
    return jnp.maximum(v, 0.0)


def _stats(sums, count):
    s = jnp.sum(sums, axis=(0, 1))
    m = s[0] / count
    v = s[1] / count - m * m
    return jnp.stack([m, lax.rsqrt(v + 1e-5)]).reshape(1, 2)


def _tc_in_body(x_ref, w_ref, b_ref, u_ref, s_ref):
    xb = x_ref[...]
    u = _relu(xb[:, :, None] * w_ref[0][None, None, :]
              + b_ref[0][None, None, :])
    u_ref[...] = u
    s1 = jnp.sum(u, axis=(0, 2))
    s2 = jnp.sum(u * u, axis=(0, 2))
    s_ref[0] = jnp.stack([s1, s2], axis=-1)


def _tc_in(x, w, b):
    return pl.pallas_call(
        _tc_in_body,
        grid=(_G,),
        in_specs=[
            pl.BlockSpec((_NB, _T), lambda i: (i, 0)),
            pl.BlockSpec((1, _C), lambda i: (0, 0)),
            pl.BlockSpec((1, _C), lambda i: (0, 0)),
        ],
        out_specs=[
            pl.BlockSpec((_NB, _T, _C), lambda i: (i, 0, 0)),
            pl.BlockSpec((1, _T, 2), lambda i: (i, 0, 0)),
        ],
        out_shape=[
            jax.ShapeDtypeStruct((_N, _T, _C), _f32),
            jax.ShapeDtypeStruct((_G, _T, 2), _f32),
        ],
    )(x, w, b)


def _tc_aspp_body(u_ref, m_ref, r_ref, lw_ref, lb_ref,
                  w02, w12, bb2, w04, w14, bb4, w08, w18, bb8,
                  ow_ref, ob_ref, h_ref, s_ref):
    u = u_ref[...]
    m = m_ref[0]
    r = r_ref[0]
    hn = ((u - m[None, :, None]) * r[None, :, None]
          * lw_ref[...][:, None, :] + lb_ref[...][:, None, :])
    parts = [hn]
    for d, w0, w1, bb in ((2, w02, w12, bb2), (4, w04, w14, bb4),
                          (8, w08, w18, bb8)):
        tl = _T - d
        a = hn[:, :tl, :].reshape(_NBA * tl, _C)
        c = hn[:, d:, :].reshape(_NBA * tl, _C)
        y = (jnp.dot(a, w0[...], preferred_element_type=_f32, precision=lax.Precision.HIGHEST)
             + jnp.dot(c, w1[...], preferred_element_type=_f32, precision=lax.Precision.HIGHEST)
             + bb[0][None, :])
        parts.append(_relu(y).reshape(_NBA, tl, _C))
    hc = jnp.concatenate(parts, axis=1).reshape(_NBA, _TCAT * _C)
    o = _relu(jnp.dot(hc, ow_ref[...], preferred_element_type=_f32, precision=lax.Precision.HIGHEST)
              + ob_ref[0][None, :])
    h_ref[...] = o
    s_ref[...] = jnp.stack([jnp.sum(o), jnp.sum(o * o)]).reshape(1, 1, 2)


def _tc_aspp(u, mt, rt, lw, lb, tw, ow, ob):
    sm16 = pl.BlockSpec((_C, _C), lambda i: (0, 0))
    b16 = pl.BlockSpec((1, _C), lambda i: (0, 0))
    return pl.pallas_call(
        _tc_aspp_body,
        grid=(_GA,),
        in_specs=[
            pl.BlockSpec((_NBA, _T, _C), lambda i: (i, 0, 0)),
            pl.BlockSpec((1, _T), lambda i: (0, 0)),
            pl.BlockSpec((1, _T), lambda i: (0, 0)),
            pl.BlockSpec((_NBA, _C), lambda i: (i, 0)),
            pl.BlockSpec((_NBA, _C), lambda i: (i, 0)),
            sm16, sm16, b16, sm16, sm16, b16, sm16, sm16, b16,
            pl.BlockSpec((_TCAT * _C, 32), lambda i: (0, 0)),
            pl.BlockSpec((1, 32), lambda i: (0, 0)),
        ],
        out_specs=[
            pl.BlockSpec((_NBA, 32), lambda i: (i, 0)),
            pl.BlockSpec((1, 1, 2), lambda i: (i, 0, 0)),
        ],
        out_shape=[
            jax.ShapeDtypeStruct((_N, 32), _f32),
            jax.ShapeDtypeStruct((_GA, 1, 2), _f32),
        ],
    )(u, mt, rt, lw, lb, *tw, ow, ob)


def _tc_msg_body(split, h_ref, sc_ref, lw_ref, lb_ref, wm_ref, ws_ref, *outs):
    m = sc_ref[0, 0]
    r = sc_ref[0, 1]
    h = (h_ref[...] - m) * r * lw_ref[...] + lb_ref[...]
    z = jnp.dot(h, wm_ref[...], preferred_element_type=_f32, precision=lax.Precision.HIGHEST)
    s = jnp.dot(h, ws_ref[...], preferred_element_type=_f32, precision=lax.Precision.HIGHEST)
    if split:
        outs[0][...] = z[:, :32]
        outs[1][...] = z[:, 32:]
        outs[2][...] = s
    else:
        outs[0][...] = z
        outs[1][...] = s


def _tc_msg(hpre, scal, lw, lb, wm, ws, cin, cout):
    split = cout == 64
    zw = 32 if split else cout
    zspecs = [pl.BlockSpec((_NB, zw), lambda i: (i, 0))] * (2 if split else 1)
    zshapes = [jax.ShapeDtypeStruct((_N, zw), _f32)] * (2 if split else 1)
    return pl.pallas_call(
        functools.partial(_tc_msg_body, split),
        grid=(_G,),
        in_specs=[
            pl.BlockSpec((_NB, cin), lambda i: (i, 0)),
            pl.BlockSpec((1, 2), lambda i: (0, 0)),
            pl.BlockSpec((_NB, cin), lambda i: (i, 0)),
            pl.BlockSpec((_NB, cin), lambda i: (i, 0)),
            pl.BlockSpec((cin, cout), lambda i: (0, 0)),
            pl.BlockSpec((cin, cout), lambda i: (0, 0)),
        ],
        out_specs=zspecs + [pl.BlockSpec((_NB, cout), lambda i: (i, 0))],
        out_shape=zshapes + [jax.ShapeDtypeStruct((_N, cout), _f32)],
    )(hpre, scal, lw, lb, wm, ws)


def _tc_comb_body(split, s_ref, *refs):
    if split:
        agg_a, agg_b, a2_ref, we_ref, b_ref, h_ref, sums_ref = refs
        agg = jnp.concatenate([agg_a[0] + agg_a[1], agg_b[0] + agg_b[1]],
                              axis=1)
    else:
        agg_a, a2_ref, we_ref, b_ref, h_ref, sums_ref = refs
        agg = agg_a[0] + agg_a[1]
    a = a2_ref[0] + a2_ref[1]
    ea = jnp.dot(a, we_ref[...], preferred_element_type=_f32, precision=lax.Precision.HIGHEST)
    h = _relu(s_ref[...] + agg + ea + b_ref[0][None, :])
    h_ref[...] = h
    sums_ref[...] = jnp.stack([jnp.sum(h), jnp.sum(h * h)]).reshape(1, 1, 2)


def _tc_comb(s, aggs, a2, wep, b, cout):
    split = cout == 64
    zw = 32 if split else cout
    aspec = [pl.BlockSpec((2, _NB, zw), lambda i: (0, i, 0))] * len(aggs)
    return pl.pallas_call(
        functools.partial(_tc_comb_body, split),
        grid=(_G,),
        in_specs=[pl.BlockSpec((_NB, cout), lambda i: (i, 0))] + aspec + [
            pl.BlockSpec((2, _NB, 8), lambda i: (0, i, 0)),
            pl.BlockSpec((8, cout), lambda i: (0, 0)),
            pl.BlockSpec((1, cout), lambda i: (0, 0)),
        ],
        out_specs=[
            pl.BlockSpec((_NB, cout), lambda i: (i, 0)),
            pl.BlockSpec((1, 1, 2), lambda i: (i, 0, 0)),
        ],
        out_shape=[
            jax.ShapeDtypeStruct((_N, cout), _f32),
            jax.ShapeDtypeStruct((_G, 1, 2), _f32),
        ],
    )(s, *aggs, a2, wep, b)


def _tc_out1_body(h_ref, sc_ref, lw_ref, lb_ref, w_ref, b_ref, o_ref, s_ref):
    m = sc_ref[0, 0]
    r = sc_ref[0, 1]
    h = (h_ref[...] - m) * r * lw_ref[...] + lb_ref[...]
    o = jnp.dot(h, w_ref[...], preferred_element_type=_f32, precision=lax.Precision.HIGHEST) + b_ref[0][None, :]
    o_ref[...] = o
    s_ref[...] = jnp.stack([jnp.sum(o), jnp.sum(o * o)]).reshape(1, 1, 2)


def _tc_out1(hpre, scal, lw, lb, w1, b1):
    return pl.pallas_call(
        _tc_out1_body,
        grid=(_G,),
        in_specs=[
            pl.BlockSpec((_NB, _C), lambda i: (i, 0)),
            pl.BlockSpec((1, 2), lambda i: (0, 0)),
            pl.BlockSpec((_NB, _C), lambda i: (i, 0)),
            pl.BlockSpec((_NB, _C), lambda i: (i, 0)),
            pl.BlockSpec((_C, _C), lambda i: (0, 0)),
            pl.BlockSpec((1, _C), lambda i: (0, 0)),
        ],
        out_specs=[
            pl.BlockSpec((_NB, _C), lambda i: (i, 0)),
            pl.BlockSpec((1, 1, 2), lambda i: (i, 0, 0)),
        ],
        out_shape=[
            jax.ShapeDtypeStruct((_N, _C), _f32),
            jax.ShapeDtypeStruct((_G, 1, 2), _f32),
        ],
    )(hpre, scal, lw, lb, w1, b1)


def _tc_out2_body(h_ref, sc_ref, lw_ref, lb_ref, w2_ref, b2_ref,
                  wf_ref, bf_ref, o_ref):
    m = sc_ref[0, 0]
    r = sc_ref[0, 1]
    h = (h_ref[...] - m) * r * lw_ref[...] + lb_ref[...]
    h = jnp.dot(h, w2_ref[...], preferred_element_type=_f32, precision=lax.Precision.HIGHEST) + b2_ref[0][None, :]
    o_ref[...] = (jnp.dot(h, wf_ref[...], preferred_element_type=_f32, precision=lax.Precision.HIGHEST)
                  + bf_ref[0][None, :])


def _tc_out2(o1, scal, lw, lb, w2, b2, wf, bf):
    return pl.pallas_call(
        _tc_out2_body,
        grid=(_G,),
        in_specs=[
            pl.BlockSpec((_NB, _C), lambda i: (i, 0)),
            pl.BlockSpec((1, 2), lambda i: (0, 0)),
            pl.BlockSpec((_NB, _C), lambda i: (i, 0)),
            pl.BlockSpec((_NB, _C), lambda i: (i, 0)),
            pl.BlockSpec((_C, _C), lambda i: (0, 0)),
            pl.BlockSpec((1, _C), lambda i: (0, 0)),
            pl.BlockSpec((_C, 8), lambda i: (0, 0)),
            pl.BlockSpec((1, 8), lambda i: (0, 0)),
        ],
        out_specs=pl.BlockSpec((_NB, 8), lambda i: (i, 0)),
        out_shape=jax.ShapeDtypeStruct((_N, 8), _f32),
    )(o1, scal, lw, lb, w2, b2, wf, bf)


def _tc_pad8_body(a_ref, o_ref):
    o_ref[...] = jnp.concatenate(
        [a_ref[...], jnp.zeros((_E // 100, 4), _f32)], axis=1)


def _tc_pad8(edge_attr):
    return pl.pallas_call(
        _tc_pad8_body,
        grid=(100,),
        in_specs=[pl.BlockSpec((_E // 100, 4), lambda i: (i, 0))],
        out_specs=pl.BlockSpec((_E // 100, 8), lambda i: (i, 0)),
        out_shape=jax.ShapeDtypeStruct((_E, 8), _f32),
    )(edge_attr)


_MAC = 2
_MB = _MAC * _CH
_NMAC = _EPW // _MB


def _make_sc_scatter(n_tables, d):
    mesh = plsc.VectorSubcoreMesh(core_axis_name="c", subcore_axis_name="s")
    out_type = [jax.ShapeDtypeStruct((2, _NTAB, d), _f32)
                for _ in range(n_tables)]
    scratch = [
        pltpu.VMEM((_MAC, _CH), jnp.int32),
        pltpu.VMEM((_MAC, _CH), jnp.int32),
        pltpu.VMEM((_MAC, _CH), jnp.int32),
        pltpu.VMEM((_MAC, _CH), jnp.int32),
        pltpu.VMEM((_MB, d), _f32),
        pltpu.VMEM((_MB, d), _f32),
        pltpu.VMEM((_ZROWS, d), _f32),
        pltpu.VMEM_SHARED((_NTAB, d), _f32),
        pltpu.SemaphoreType.DMA,
        pltpu.SemaphoreType.DMA,
    ]

    @functools.partial(pl.kernel, out_type=out_type, mesh=mesh,
                       scratch_types=scratch,
                       compiler_params=pltpu.CompilerParams(
                           use_tc_tiling_on_sc=False))
    def sc_kernel(*refs):
        tables = refs[:n_tables]
        src_hbm, dst_hbm, z_hbm = refs[n_tables:n_tables + 3]
        outs = refs[n_tables + 3:2 * n_tables + 3]
        (sidx0, sidx1, didx0, didx1, rows0, rows1, zero_v, tab_sh,
         semg0, semg1) = refs[2 * n_tables + 3:]
        cid = lax.axis_index("c")
        sid = lax.axis_index("s")
        wid = cid * 16 + sid
        crow = wid * _NCHUNK
        pltpu.sync_copy(z_hbm, zero_v)

        maxm = _NMAC - 1

        def load_idx(m, sbuf, dbuf):
            r = pl.multiple_of(
                jnp.minimum(crow + m * _MAC, crow + maxm * _MAC), 2)
            pltpu.sync_copy(src_hbm.at[pl.ds(r, _MAC)], sbuf)
            pltpu.sync_copy(dst_hbm.at[pl.ds(r, _MAC)], dbuf)

        def fire(tab, sbuf, rows, sem):
            descs = [pltpu.async_copy(tab.at[sbuf.at[j]],
                                      rows.at[pl.ds(j * _CH, _CH)], sem)
                     for j in range(_MAC)]
            for dsc in descs:
                dsc.wait()

        def scatter(dbuf, rows):
            for j in range(_MAC):
                pltpu.sync_copy(rows.at[pl.ds(j * _CH, _CH)],
                                tab_sh.at[dbuf.at[j]], add=True)

        for p in range(n_tables):
            tab = tables[p]

            def zbody(i, _, sid=sid):
                off = pl.multiple_of(sid * 3200 + i * _ZROWS, 8)
                pltpu.sync_copy(zero_v, tab_sh.at[pl.ds(off, _ZROWS)])
                return 0
            lax.fori_loop(0, 3200 // _ZROWS, zbody, 0)
            plsc.subcore_barrier()

            def qbody(m, _, tab=tab):
                load_idx(m, sidx0, didx0)
                fire(tab, sidx0, rows0, semg0)
                scatter(didx0, rows0)
                return 0
            lax.fori_loop(0, _NMAC, qbody, 0)
            plsc.subcore_barrier()

            off = pl.multiple_of(sid * _ROWS_OUT, 8)
            pltpu.sync_copy(tab_sh.at[pl.ds(off, _ROWS_OUT)],
                            outs[p].at[cid, pl.ds(off, _ROWS_OUT)])
            if p < n_tables - 1:
                plsc.subcore_barrier()

    return sc_kernel


_sc_cache = {}


def _sc_scatter(n_tables, d, *args):
    key = (n_tables, d)
    if key not in _sc_cache:
        _sc_cache[key] = _make_sc_scatter(n_tables, d)
    return _sc_cache[key](*args)


def kernel(x, edge_index, edge_attr, params):
    p = params

    inw = p['in_W'].reshape(1, _C)
    inb = p['in_b'].reshape(1, _C)
    tw = []
    for d in _DIAS:
        w = p['t%d_W' % d]
        tw += [w[:, :, 0].T, w[:, :, 1].T, p['t%d_b' % d].reshape(1, _C)]
    ow = p['out_W'].transpose(2, 1, 0).reshape(_TCAT * _C, 32)
    ob = p['out_b'].reshape(1, 32)

    npad = _EPAD - _E
    padi = jnp.arange(npad, dtype=jnp.int32)
    src_p = jnp.concatenate([edge_index[0], padi % jnp.int32(_N)])
    dst_p = jnp.concatenate([edge_index[1], jnp.int32(_N) + (padi % 1024)])
    iota_e = jnp.arange(_EPAD, dtype=jnp.int32) % jnp.int32(_E)
    dst2 = dst_p.reshape(-1, _CH)
    src2 = src_p.reshape(-1, _CH)
    iot2 = iota_e.reshape(-1, _CH)
    attr8 = _tc_pad8(edge_attr)
    z32 = jnp.zeros((_ZROWS, 32), _f32)
    z16 = jnp.zeros((_ZROWS, 16), _f32)
    z8 = jnp.zeros((_ZROWS, 8), _f32)

    (a2,) = _sc_scatter(1, 8, attr8, iot2, dst2, z8)

    u, sums_t = _tc_in(x, inw, inb)
    st = jnp.sum(sums_t, axis=0)
    cnt = jnp.float32(_N * _C)
    mt = st[:, 0] / cnt
    vt = st[:, 1] / cnt - mt * mt
    h1pre, s1 = _tc_aspp(u, mt.reshape(1, _T),
                         lax.rsqrt(vt + 1e-5).reshape(1, _T),
                         p['ln_in_w'], p['ln_in_b'], tw, ow, ob)

    scal = _stats(s1, _N * 32.0)
    z0a, z0b, s0 = _tc_msg(h1pre, scal, p['ln_out_w'], p['ln_out_b'],
                           p['s0_Wmsg'], p['s0_Wself'], 32, 64)
    agg0a, agg0b = _sc_scatter(2, 32, z0a, z0b, src2, dst2, z32)
    wep0 = jnp.zeros((8, 64), _f32).at[:4].set(p['s0_We'])
    h2pre, s2 = _tc_comb(s0, [agg0a, agg0b], a2, wep0,
                         p['s0_b'].reshape(1, 64), 64)

    scal = _stats(s2, _N * 64.0)
    z1, s1s = _tc_msg(h2pre, scal, p['n0_w'], p['n0_b'],
                      p['s1_Wmsg'], p['s1_Wself'], 64, 32)
    (agg1,) = _sc_scatter(1, 32, z1, src2, dst2, z32)
    wep1 = jnp.zeros((8, 32), _f32).at[:4].set(p['s1_We'])
    h3pre, s3 = _tc_comb(s1s, [agg1], a2, wep1, p['s1_b'].reshape(1, 32), 32)

    scal = _stats(s3, _N * 32.0)
    z2, s2s = _tc_msg(h3pre, scal, p['n1_w'], p['n1_b'],
                      p['s2_Wmsg'], p['s2_Wself'], 32, 16)
    (agg2,) = _sc_scatter(1, 16, z2, src2, dst2, z16)
    wep2 = jnp.zeros((8, 16), _f32).at[:4].set(p['s2_We'])
    h4pre, s4 = _tc_comb(s2s, [agg2], a2, wep2, p['s2_b'].reshape(1, 16), 16)

    scal = _stats(s4, _N * 16.0)
    o1pre, s5 = _tc_out1(h4pre, scal, p['n2_w'], p['n2_b'],
                         p['o_W1'], p['o_b1'].reshape(1, _C))
    scal = _stats(s5, _N * 16.0)
    wf = jnp.zeros((_C, 8), _f32).at[:, :1].set(p['o_Wfc'])
    bf = jnp.zeros((1, 8), _f32).at[0, 0].set(p['o_bfc'][0])
    o = _tc_out2(o1pre, scal, p['o_lnw'], p['o_lnb'],
                 p['o_W2'], p['o_b2'].reshape(1, _C), wf, bf)
    return o[:, :1].reshape(1, _N, 1)
